# baseline (speedup 1.0000x reference)
import functools

import jax
import jax.numpy as jnp
from jax import lax
from jax.experimental import pallas as pl
from jax.experimental.pallas import tpu as pltpu
from jax.experimental.pallas import tpu_sc as plsc


def _make_broadcast_kernel(B, T, D, t):
    info = plsc.get_sparse_core_info()
    nc, ns, nl = info.num_cores, info.num_subcores, info.num_lanes
    nw = nc * ns
    rows_per_w = B // nw
    blk = 32
    n_dma = rows_per_w // blk
    mesh = plsc.VectorSubcoreMesh(core_axis_name="c", subcore_axis_name="s")

    @functools.partial(
        pl.kernel,
        mesh=mesh,
        out_type=jax.ShapeDtypeStruct((B, D), jnp.float32),
        scratch_types=[
            pltpu.VMEM((blk, D), jnp.float32),
            pltpu.SemaphoreType.DMA,
        ],
    )
    def body(param_hbm, out_hbm, buf, sem):
        wid = lax.axis_index("s") * nc + lax.axis_index("c")
        base = wid * rows_per_w
        pltpu.sync_copy(param_hbm.at[pl.ds(t, 1)], buf.at[pl.ds(0, 1)])
        chunks = [buf[0, pl.ds(i * nl, nl)] for i in range(D // nl)]
        for r in range(1, blk):
            for i in range(D // nl):
                buf[r, pl.ds(i * nl, nl)] = chunks[i]
        copies = [
            pltpu.async_copy(buf, out_hbm.at[pl.ds(base + j * blk, blk)], sem)
            for j in range(n_dma)
        ]
        for c in copies:
            c.wait()

    return body


def kernel(designs, outcomes, designs_param):
    B = designs.shape[0]
    t = designs.shape[1]
    T, D = designs_param.shape
    return _make_broadcast_kernel(B, T, D, t)(designs_param)


# speedup vs baseline: 1.0092x; 1.0092x over previous
import functools

import jax
import jax.numpy as jnp
from jax import lax
from jax.experimental import pallas as pl
from jax.experimental.pallas import tpu as pltpu
from jax.experimental.pallas import tpu_sc as plsc


def _make_broadcast_kernel(B, T, D, t):
    info = plsc.get_sparse_core_info()
    nc, ns, nl = info.num_cores, info.num_subcores, info.num_lanes
    nw = nc * ns
    rows_per_w = B // nw
    blk = 16
    n_dma = rows_per_w // blk
    mesh = plsc.VectorSubcoreMesh(core_axis_name="c", subcore_axis_name="s")

    @functools.partial(
        pl.kernel,
        mesh=mesh,
        out_type=jax.ShapeDtypeStruct((B, D), jnp.float32),
        scratch_types=[
            pltpu.VMEM((blk, D), jnp.float32),
            pltpu.SemaphoreType.DMA,
        ],
    )
    def body(param_hbm, out_hbm, buf, sem):
        wid = lax.axis_index("s") * nc + lax.axis_index("c")
        base = wid * rows_per_w
        pltpu.sync_copy(param_hbm.at[pl.ds(t, 1)], buf.at[pl.ds(0, 1)])
        chunks = [buf[0, pl.ds(i * nl, nl)] for i in range(D // nl)]
        for r in range(1, blk):
            for i in range(D // nl):
                buf[r, pl.ds(i * nl, nl)] = chunks[i]
        copies = [
            pltpu.async_copy(buf, out_hbm.at[pl.ds(base + j * blk, blk)], sem)
            for j in range(n_dma)
        ]
        for c in copies:
            c.wait()

    return body


def kernel(designs, outcomes, designs_param):
    B = designs.shape[0]
    t = designs.shape[1]
    T, D = designs_param.shape
    return _make_broadcast_kernel(B, T, D, t)(designs_param)


# speedup vs baseline: 1.1391x; 1.1287x over previous
import functools

import jax
import jax.numpy as jnp
from jax import lax
from jax.experimental import pallas as pl
from jax.experimental.pallas import tpu as pltpu
from jax.experimental.pallas import tpu_sc as plsc


def _make_broadcast_kernel(B, T, D, t):
    info = plsc.get_sparse_core_info()
    nc, ns, nl = info.num_cores, info.num_subcores, info.num_lanes
    nw = nc * ns
    rows_per_w = B // nw
    blk = 16
    n_dma = rows_per_w // blk
    mesh = plsc.VectorSubcoreMesh(core_axis_name="c", subcore_axis_name="s", num_cores=1)

    @functools.partial(
        pl.kernel,
        mesh=mesh,
        out_type=jax.ShapeDtypeStruct((B, D), jnp.float32),
        scratch_types=[
            pltpu.VMEM((blk, D), jnp.float32),
            pltpu.SemaphoreType.DMA,
        ],
    )
    def body(param_hbm, out_hbm, buf, sem):
        wid = lax.axis_index("s") * nc + lax.axis_index("c")
        base = wid * rows_per_w
        pltpu.sync_copy(param_hbm.at[pl.ds(t, 1)], buf.at[pl.ds(0, 1)])
        chunks = [buf[0, pl.ds(i * nl, nl)] for i in range(D // nl)]
        for r in range(1, blk):
            for i in range(D // nl):
                buf[r, pl.ds(i * nl, nl)] = chunks[i]
        copies = [
            pltpu.async_copy(buf, out_hbm.at[pl.ds(base + j * blk, blk)], sem)
            for j in range(n_dma)
        ]
        for c in copies:
            c.wait()

    return body


def kernel(designs, outcomes, designs_param):
    B = designs.shape[0]
    t = designs.shape[1]
    T, D = designs_param.shape
    return _make_broadcast_kernel(B, T, D, t)(designs_param)
